# baseline (speedup 1.0000x reference)
import functools

import jax
import jax.numpy as jnp
from jax import lax
from jax.experimental import pallas as pl
from jax.experimental.pallas import tpu as pltpu
from jax.experimental.pallas import tpu_sc as plsc

HIDDEN = 2048
INTER = 1408
K = 4
L = 16
RC = 8
GU_CH = 176 // RC
DN_CH = 256 // RC

_LANES = None


def _dot8(buf, xv, ncol):
    def body(cc, accs):
        xc = xv[pl.ds(cc * L, L)]
        return tuple(accs[r] + buf[r, pl.ds(cc * L, L)] * xc for r in range(RC))

    init = tuple(jnp.zeros((L,), jnp.float32) for _ in range(RC))
    accs = lax.fori_loop(0, ncol // L, body, init, unroll=False)
    return [jnp.sum(accs[r]) for r in range(RC)]


def _insert8(vec, scalars, lane0, lane_iota):
    for r, s in enumerate(scalars):
        vec = jnp.where(lane_iota == (lane0 + r), s, vec)
    return vec


def _sc_body(xr, gr, ur, dr, guir, dnir, wr, outr,
             xv, giv, div, w16v, gbuf0, gbuf1, ubuf0, ubuf1,
             dbuf0, dbuf1, interbuf, interfull, obuf, shared,
             gsem0, gsem1, usem0, usem1):
    c = lax.axis_index("c")
    s = lax.axis_index("s")
    slot_loc = s // 8
    col8 = s % 8
    lane_iota = lax.iota(jnp.int32, L)

    pltpu.sync_copy(xr, xv)
    cb_gu = c * 352 + slot_loc * 176 + col8 * 22
    pltpu.sync_copy(guir.at[pl.ds(cb_gu * RC, 176)], giv)
    cb_dn = c * 512 + s * 32
    pltpu.sync_copy(dnir.at[pl.ds(cb_dn * RC, 256)], div)
    pltpu.sync_copy(wr, w16v)
    slot_glob = 2 * c + slot_loc
    wall = w16v[pl.ds(0, L)]
    wscal = jnp.sum(jnp.where(lane_iota == slot_glob, wall, 0.0))
    wvec = jnp.full((L,), wscal, jnp.float32)

    gbufs = (gbuf0, gbuf1)
    ubufs = (ubuf0, ubuf1)
    gsems = (gsem0, gsem1)
    usems = (usem0, usem1)

    def start_gu(k, p):
        kk = jnp.minimum(k, GU_CH - 1)

        @pl.when(k < GU_CH)
        def _():
            pltpu.async_copy(gr.at[giv.at[pl.ds(kk * RC, RC)]], gbufs[p], gsems[p])
            pltpu.async_copy(ur.at[giv.at[pl.ds(kk * RC, RC)]], ubufs[p], usems[p])

    start_gu(0, 0)
    start_gu(1, 1)

    def gu_pair(i, _):
        gsc = []
        usc = []
        for p in range(2):
            k = 2 * i + p
            pltpu.make_async_copy(gr.at[giv.at[pl.ds(0, RC)]], gbufs[p], gsems[p]).wait()
            gsc += _dot8(gbufs[p], xv, HIDDEN)
            pltpu.make_async_copy(ur.at[giv.at[pl.ds(0, RC)]], ubufs[p], usems[p]).wait()
            usc += _dot8(ubufs[p], xv, HIDDEN)
            start_gu(k + 2, p)
        gvec = jnp.zeros((L,), jnp.float32)
        uvec = jnp.zeros((L,), jnp.float32)
        gvec = _insert8(gvec, gsc[:8], 0, lane_iota)
        gvec = _insert8(gvec, gsc[8:], 8, lane_iota)
        uvec = _insert8(uvec, usc[:8], 0, lane_iota)
        uvec = _insert8(uvec, usc[8:], 8, lane_iota)
        sig = 1.0 / (1.0 + jnp.exp(-gvec))
        ivec = gvec * sig * uvec * wvec
        interbuf[pl.ds(i * L, L)] = ivec
        return 0

    lax.fori_loop(0, GU_CH // 2, gu_pair, 0, unroll=False)

    pltpu.sync_copy(interbuf, shared.at[pl.ds(slot_loc * INTER + col8 * 176, 176)])
    plsc.subcore_barrier()
    pltpu.sync_copy(shared.at[pl.ds(slot_loc * INTER, INTER)], interfull)

    dbufs = (dbuf0, dbuf1)

    def start_dn(k, p):
        kk = jnp.minimum(k, DN_CH - 1)

        @pl.when(k < DN_CH)
        def _():
            pltpu.async_copy(dr.at[div.at[pl.ds(kk * RC, RC)]], dbufs[p], gsems[p])

    start_dn(0, 0)
    start_dn(1, 1)

    def dn_pair(i, _):
        dsc = []
        for p in range(2):
            k = 2 * i + p
            pltpu.make_async_copy(dr.at[div.at[pl.ds(0, RC)]], dbufs[p], gsems[p]).wait()
            dsc += _dot8(dbufs[p], interfull, INTER)
            start_dn(k + 2, p)
        dvec = jnp.zeros((L,), jnp.float32)
        dvec = _insert8(dvec, dsc[:8], 0, lane_iota)
        dvec = _insert8(dvec, dsc[8:], 8, lane_iota)
        obuf[pl.ds(i * L, L)] = dvec
        return 0

    lax.fori_loop(0, DN_CH // 2, dn_pair, 0, unroll=False)

    pltpu.sync_copy(obuf, outr.at[slot_glob, pl.ds(col8 * 256, 256)])


@jax.jit
def _run_sc(x_flat, gate_rows, up_rows, down_rows, gu_idx3, dn_idx3, w16):
    mesh = plsc.VectorSubcoreMesh(core_axis_name="c", subcore_axis_name="s")
    fn = pl.kernel(
        _sc_body,
        out_type=jax.ShapeDtypeStruct((K, HIDDEN), jnp.float32),
        mesh=mesh,
        compiler_params=pltpu.CompilerParams(needs_layout_passes=False),
        scratch_types=[
            pltpu.VMEM((HIDDEN,), jnp.float32),
            pltpu.VMEM((176,), jnp.int32),
            pltpu.VMEM((256,), jnp.int32),
            pltpu.VMEM((L,), jnp.float32),
            pltpu.VMEM((RC, HIDDEN), jnp.float32),
            pltpu.VMEM((RC, HIDDEN), jnp.float32),
            pltpu.VMEM((RC, HIDDEN), jnp.float32),
            pltpu.VMEM((RC, HIDDEN), jnp.float32),
            pltpu.VMEM((RC, INTER), jnp.float32),
            pltpu.VMEM((RC, INTER), jnp.float32),
            pltpu.VMEM((176,), jnp.float32),
            pltpu.VMEM((INTER,), jnp.float32),
            pltpu.VMEM((256,), jnp.float32),
            pltpu.VMEM_SHARED((2 * INTER,), jnp.float32),
            pltpu.SemaphoreType.DMA,
            pltpu.SemaphoreType.DMA,
            pltpu.SemaphoreType.DMA,
            pltpu.SemaphoreType.DMA,
        ],
    )
    return fn(x_flat, gate_rows, up_rows, down_rows, gu_idx3, dn_idx3, w16)


def kernel(x_bc1t, topk_idx, topk_weights, gate_proj_all, up_proj_all, down_proj_all):
    x_flat = x_bc1t.reshape(HIDDEN)
    idx = topk_idx.astype(jnp.int32)
    gu_idx = (idx[:, None] * INTER + jnp.arange(INTER, dtype=jnp.int32)[None, :])
    dn_idx = (idx[:, None] * HIDDEN + jnp.arange(HIDDEN, dtype=jnp.int32)[None, :])
    gu_idx3 = gu_idx.reshape(K * INTER)
    dn_idx3 = dn_idx.reshape(K * HIDDEN)
    w16 = jnp.zeros((L,), jnp.float32).at[:K].set(topk_weights)
    gate_rows = gate_proj_all.reshape(-1, HIDDEN)
    up_rows = up_proj_all.reshape(-1, HIDDEN)
    down_rows = down_proj_all.reshape(-1, INTER)
    partial = _run_sc(x_flat, gate_rows, up_rows, down_rows, gu_idx3, dn_idx3, w16)
    return partial.sum(axis=0).reshape(1, HIDDEN, 1, 1)


# speedup vs baseline: 1.2833x; 1.2833x over previous
import jax
import jax.numpy as jnp
from jax import lax
from jax.experimental import pallas as pl
from jax.experimental.pallas import tpu as pltpu
from jax.experimental.pallas import tpu_sc as plsc

HIDDEN = 2048
INTER = 1408
K = 4
L = 16
RC = 8
GU_CH = 88 // RC
DN_CH = 128 // RC
SC_SLOT0 = 2
IB = 128
NB = INTER // IB
K_TC = SC_SLOT0


def _tc_body(idx_ref, w_ref, x_ref, gate_ref, up_ref, down_ref, out_ref):
    e = pl.program_id(0)
    ib = pl.program_id(1)

    @pl.when(jnp.logical_and(e == 0, ib == 0))
    def _init():
        out_ref[...] = jnp.zeros_like(out_ref)

    x = x_ref[...]
    g = gate_ref[0]
    u = up_ref[0]
    d = down_ref[0]

    dn = (((1,), (1,)), ((), ()))
    gate_out = jax.lax.dot_general(x, g, dn, preferred_element_type=jnp.float32)
    up_out = jax.lax.dot_general(x, u, dn, preferred_element_type=jnp.float32)
    inter = jax.nn.silu(gate_out) * up_out
    inter = inter * w_ref[e]
    partial = jax.lax.dot_general(inter, d, dn, preferred_element_type=jnp.float32)
    out_ref[...] += partial


def _run_tc(x_row, topk_idx, topk_weights, gate_proj_all, up_proj_all, down_proj_all):
    grid_spec = pltpu.PrefetchScalarGridSpec(
        num_scalar_prefetch=2,
        grid=(K_TC, NB),
        in_specs=[
            pl.BlockSpec((1, HIDDEN), lambda e, ib, idx, w: (0, 0)),
            pl.BlockSpec((1, IB, HIDDEN), lambda e, ib, idx, w: (idx[e], ib, 0)),
            pl.BlockSpec((1, IB, HIDDEN), lambda e, ib, idx, w: (idx[e], ib, 0)),
            pl.BlockSpec((1, HIDDEN, IB), lambda e, ib, idx, w: (idx[e], 0, ib)),
        ],
        out_specs=pl.BlockSpec((1, HIDDEN), lambda e, ib, idx, w: (0, 0)),
    )
    return pl.pallas_call(
        _tc_body,
        grid_spec=grid_spec,
        out_shape=jax.ShapeDtypeStruct((1, HIDDEN), jnp.float32),
        compiler_params=pltpu.CompilerParams(
            dimension_semantics=("arbitrary", "arbitrary"),
        ),
    )(topk_idx, topk_weights, x_row, gate_proj_all, up_proj_all, down_proj_all)


def _dot8(buf, xv, ncol):
    def body(cc, accs):
        xc = xv[pl.ds(cc * L, L)]
        return tuple(accs[r] + buf[r, pl.ds(cc * L, L)] * xc for r in range(RC))

    init = tuple(jnp.zeros((L,), jnp.float32) for _ in range(RC))
    accs = lax.fori_loop(0, ncol // L, body, init, unroll=False)
    return [jnp.sum(accs[r]) for r in range(RC)]


def _insert8(vec, scalars, lane0, lane_iota):
    for r, s in enumerate(scalars):
        vec = jnp.where(lane_iota == (lane0 + r), s, vec)
    return vec


def _sc_body(xr, gr, ur, dr, guir, dnir, wr, outr,
             xv, giv, div, w16v, gbuf0, gbuf1, ubuf0, ubuf1,
             dbuf0, dbuf1, interbuf, interfull, obuf, shared,
             gsem0, gsem1, usem0, usem1):
    c = lax.axis_index("c")
    s = lax.axis_index("s")
    lane_iota = lax.iota(jnp.int32, L)
    slot_glob = SC_SLOT0 + c

    pltpu.sync_copy(xr, xv)
    cb_gu = slot_glob * INTER + s * 88
    pltpu.sync_copy(guir.at[pl.ds(cb_gu, 88)], giv)
    cb_dn = slot_glob * HIDDEN + s * 128
    pltpu.sync_copy(dnir.at[pl.ds(cb_dn, 128)], div)
    pltpu.sync_copy(wr, w16v)
    wall = w16v[pl.ds(0, L)]
    wscal = jnp.sum(jnp.where(lane_iota == slot_glob, wall, 0.0))
    wvec = jnp.full((L,), wscal, jnp.float32)

    gbufs = (gbuf0, gbuf1)
    ubufs = (ubuf0, ubuf1)
    gsems = (gsem0, gsem1)
    usems = (usem0, usem1)

    def start_gu(k, p):
        kk = jnp.minimum(k, GU_CH - 1)

        @pl.when(k < GU_CH)
        def _():
            pltpu.async_copy(gr.at[giv.at[pl.ds(kk * RC, RC)]], gbufs[p], gsems[p])
            pltpu.async_copy(ur.at[giv.at[pl.ds(kk * RC, RC)]], ubufs[p], usems[p])

    start_gu(0, 0)
    start_gu(1, 1)

    def gu_chunk(k, p):
        pltpu.make_async_copy(gr.at[giv.at[pl.ds(0, RC)]], gbufs[p], gsems[p]).wait()
        gsc = _dot8(gbufs[p], xv, HIDDEN)
        pltpu.make_async_copy(ur.at[giv.at[pl.ds(0, RC)]], ubufs[p], usems[p]).wait()
        usc = _dot8(ubufs[p], xv, HIDDEN)
        start_gu(k + 2, p)
        gvec = _insert8(jnp.zeros((L,), jnp.float32), gsc, 0, lane_iota)
        uvec = _insert8(jnp.zeros((L,), jnp.float32), usc, 0, lane_iota)
        return gvec, uvec

    def gu_pair(i, _):
        g0, u0 = gu_chunk(2 * i, 0)

        def silu_store(gvec, uvec, off):
            sig = 1.0 / (1.0 + jnp.exp(-gvec))
            interbuf[pl.ds(off, L)] = gvec * sig * uvec * wvec

        silu_store(g0, u0, 2 * i * RC)

        @pl.when(2 * i + 1 < GU_CH)
        def _():
            g1, u1 = gu_chunk(2 * i + 1, 1)
            silu_store(g1, u1, jnp.minimum((2 * i + 1) * RC, 88 - L))
        return 0

    lax.fori_loop(0, (GU_CH + 1) // 2, gu_pair, 0, unroll=False)

    pltpu.sync_copy(interbuf.at[pl.ds(0, 88)], shared.at[pl.ds(s * 88, 88)])
    plsc.subcore_barrier()
    pltpu.sync_copy(shared, interfull)

    dbufs = (dbuf0, dbuf1)

    def start_dn(k, p):
        kk = jnp.minimum(k, DN_CH - 1)

        @pl.when(k < DN_CH)
        def _():
            pltpu.async_copy(dr.at[div.at[pl.ds(kk * RC, RC)]], dbufs[p], gsems[p])

    start_dn(0, 0)
    start_dn(1, 1)

    def dn_pair(i, _):
        dsc = []
        for p in range(2):
            k = 2 * i + p
            pltpu.make_async_copy(dr.at[div.at[pl.ds(0, RC)]], dbufs[p], gsems[p]).wait()
            dsc += _dot8(dbufs[p], interfull, INTER)
            start_dn(k + 2, p)
        dvec = _insert8(jnp.zeros((L,), jnp.float32), dsc[:8], 0, lane_iota)
        dvec = _insert8(dvec, dsc[8:], 8, lane_iota)
        obuf[pl.ds(i * L, L)] = dvec
        return 0

    lax.fori_loop(0, DN_CH // 2, dn_pair, 0, unroll=False)

    pltpu.sync_copy(obuf, outr.at[c, pl.ds(s * 128, 128)])


def _run_sc(x_flat, gate_rows, up_rows, down_rows, gu_idx, dn_idx, w16):
    mesh = plsc.VectorSubcoreMesh(core_axis_name="c", subcore_axis_name="s")
    fn = pl.kernel(
        _sc_body,
        out_type=jax.ShapeDtypeStruct((2, HIDDEN), jnp.float32),
        mesh=mesh,
        compiler_params=pltpu.CompilerParams(needs_layout_passes=False),
        scratch_types=[
            pltpu.VMEM((HIDDEN,), jnp.float32),
            pltpu.VMEM((88,), jnp.int32),
            pltpu.VMEM((128,), jnp.int32),
            pltpu.VMEM((L,), jnp.float32),
            pltpu.VMEM((RC, HIDDEN), jnp.float32),
            pltpu.VMEM((RC, HIDDEN), jnp.float32),
            pltpu.VMEM((RC, HIDDEN), jnp.float32),
            pltpu.VMEM((RC, HIDDEN), jnp.float32),
            pltpu.VMEM((RC, INTER), jnp.float32),
            pltpu.VMEM((RC, INTER), jnp.float32),
            pltpu.VMEM((96,), jnp.float32),
            pltpu.VMEM((INTER,), jnp.float32),
            pltpu.VMEM((128,), jnp.float32),
            pltpu.VMEM_SHARED((INTER,), jnp.float32),
            pltpu.SemaphoreType.DMA,
            pltpu.SemaphoreType.DMA,
            pltpu.SemaphoreType.DMA,
            pltpu.SemaphoreType.DMA,
        ],
    )
    return fn(x_flat, gate_rows, up_rows, down_rows, gu_idx, dn_idx, w16)


@jax.jit
def _run(x_bc1t, topk_idx, topk_weights, gate_proj_all, up_proj_all, down_proj_all):
    x_flat = x_bc1t.reshape(HIDDEN)
    idx = topk_idx.astype(jnp.int32)
    gu_idx = (idx[:, None] * INTER
              + jnp.arange(INTER, dtype=jnp.int32)[None, :]).reshape(K * INTER)
    dn_idx = (idx[:, None] * HIDDEN
              + jnp.arange(HIDDEN, dtype=jnp.int32)[None, :]).reshape(K * HIDDEN)
    w16 = jnp.zeros((L,), jnp.float32).at[:K].set(topk_weights)
    gate_rows = gate_proj_all.reshape(-1, HIDDEN)
    up_rows = up_proj_all.reshape(-1, HIDDEN)
    down_rows = down_proj_all.reshape(-1, INTER)
    sc_part = _run_sc(x_flat, gate_rows, up_rows, down_rows, gu_idx, dn_idx, w16)
    tc_part = _run_tc(x_flat.reshape(1, HIDDEN), idx, topk_weights,
                      gate_proj_all, up_proj_all, down_proj_all)
    out = tc_part[0] + sc_part[0] + sc_part[1]
    return out.reshape(1, HIDDEN, 1, 1)


def kernel(x_bc1t, topk_idx, topk_weights, gate_proj_all, up_proj_all, down_proj_all):
    return _run(x_bc1t, topk_idx, topk_weights, gate_proj_all,
                up_proj_all, down_proj_all)


# speedup vs baseline: 1.2835x; 1.0001x over previous
import jax
import jax.numpy as jnp
from jax import lax
from jax.experimental import pallas as pl
from jax.experimental.pallas import tpu as pltpu
from jax.experimental.pallas import tpu_sc as plsc

HIDDEN = 2048
INTER = 1408
K = 4
L = 16
RC = 8
GU_CH = 88 // RC
DN_CH = 128 // RC
SC_SLOT0 = 2
IB = 128
NB = INTER // IB
K_TC = SC_SLOT0


def _tc_body(idx_ref, w_ref, x_ref, gate_ref, up_ref, down_ref, out_ref):
    e = pl.program_id(0)
    ib = pl.program_id(1)

    @pl.when(jnp.logical_and(e == 0, ib == 0))
    def _init():
        out_ref[...] = jnp.zeros_like(out_ref)

    x = x_ref[...]
    g = gate_ref[0]
    u = up_ref[0]
    d = down_ref[0]

    dn = (((1,), (1,)), ((), ()))
    gate_out = jax.lax.dot_general(x, g, dn, preferred_element_type=jnp.float32)
    up_out = jax.lax.dot_general(x, u, dn, preferred_element_type=jnp.float32)
    inter = jax.nn.silu(gate_out) * up_out
    inter = inter * w_ref[e]
    partial = jax.lax.dot_general(inter, d, dn, preferred_element_type=jnp.float32)
    out_ref[...] += partial


def _run_tc(x_row, topk_idx, topk_weights, gate_proj_all, up_proj_all, down_proj_all):
    grid_spec = pltpu.PrefetchScalarGridSpec(
        num_scalar_prefetch=2,
        grid=(K_TC, NB),
        in_specs=[
            pl.BlockSpec((1, HIDDEN), lambda e, ib, idx, w: (0, 0)),
            pl.BlockSpec((1, IB, HIDDEN), lambda e, ib, idx, w: (idx[e], ib, 0)),
            pl.BlockSpec((1, IB, HIDDEN), lambda e, ib, idx, w: (idx[e], ib, 0)),
            pl.BlockSpec((1, HIDDEN, IB), lambda e, ib, idx, w: (idx[e], 0, ib)),
        ],
        out_specs=pl.BlockSpec((1, HIDDEN), lambda e, ib, idx, w: (0, 0)),
    )
    return pl.pallas_call(
        _tc_body,
        grid_spec=grid_spec,
        out_shape=jax.ShapeDtypeStruct((1, HIDDEN), jnp.float32),
        compiler_params=pltpu.CompilerParams(
            dimension_semantics=("arbitrary", "arbitrary"),
        ),
    )(topk_idx, topk_weights, x_row, gate_proj_all, up_proj_all, down_proj_all)


def _dot8(buf, xv, ncol):
    def body(cc, accs):
        xc = xv[pl.ds(cc * L, L)]
        return tuple(accs[r] + buf[r, pl.ds(cc * L, L)] * xc for r in range(RC))

    init = tuple(jnp.zeros((L,), jnp.float32) for _ in range(RC))
    accs = lax.fori_loop(0, ncol // L, body, init, unroll=False)
    return [jnp.sum(accs[r]) for r in range(RC)]


def _insert8(vec, scalars, lane0, lane_iota):
    for r, s in enumerate(scalars):
        vec = jnp.where(lane_iota == (lane0 + r), s, vec)
    return vec


def _sc_body(xr, gr, ur, dr, guir, dnir, wr, outr,
             xv, giv, div, w16v, gbuf0, gbuf1, ubuf0, ubuf1,
             dbuf0, dbuf1, interbuf, interfull, obuf, shared,
             gsem0, gsem1, usem0, usem1):
    c = lax.axis_index("c")
    s = lax.axis_index("s")
    lane_iota = lax.iota(jnp.int32, L)
    slot_glob = SC_SLOT0 + c

    pltpu.sync_copy(xr, xv)
    cb_gu = slot_glob * INTER + s * 88
    pltpu.sync_copy(guir.at[pl.ds(cb_gu, 88)], giv)
    cb_dn = slot_glob * HIDDEN + s * 128
    pltpu.sync_copy(dnir.at[pl.ds(cb_dn, 128)], div)
    pltpu.sync_copy(wr, w16v)
    wall = w16v[pl.ds(0, L)]
    wscal = jnp.sum(jnp.where(lane_iota == slot_glob, wall, 0.0))
    wvec = jnp.full((L,), wscal, jnp.float32)

    gbufs = (gbuf0, gbuf1)
    ubufs = (ubuf0, ubuf1)
    gsems = (gsem0, gsem1)
    usems = (usem0, usem1)

    def start_gu(k, p):
        kk = jnp.minimum(k, GU_CH - 1)

        @pl.when(k < GU_CH)
        def _():
            pltpu.async_copy(gr.at[giv.at[pl.ds(kk * RC, RC)]], gbufs[p], gsems[p])
            pltpu.async_copy(ur.at[giv.at[pl.ds(kk * RC, RC)]], ubufs[p], usems[p])

    start_gu(0, 0)
    start_gu(1, 1)

    def gu_chunk(k, p):
        pltpu.make_async_copy(gr.at[giv.at[pl.ds(0, RC)]], gbufs[p], gsems[p]).wait()
        gsc = _dot8(gbufs[p], xv, HIDDEN)
        pltpu.make_async_copy(ur.at[giv.at[pl.ds(0, RC)]], ubufs[p], usems[p]).wait()
        usc = _dot8(ubufs[p], xv, HIDDEN)
        start_gu(k + 2, p)
        gvec = _insert8(jnp.zeros((L,), jnp.float32), gsc, 0, lane_iota)
        uvec = _insert8(jnp.zeros((L,), jnp.float32), usc, 0, lane_iota)
        return gvec, uvec

    def gu_pair(i, _):
        g0, u0 = gu_chunk(2 * i, 0)

        def silu_store(gvec, uvec, off):
            sig = 1.0 / (1.0 + jnp.exp(-gvec))
            interbuf[pl.ds(off, L)] = gvec * sig * uvec * wvec

        silu_store(g0, u0, 2 * i * RC)

        @pl.when(2 * i + 1 < GU_CH)
        def _():
            g1, u1 = gu_chunk(2 * i + 1, 1)
            silu_store(g1, u1, jnp.minimum((2 * i + 1) * RC, 88 - L))
        return 0

    lax.fori_loop(0, (GU_CH + 1) // 2, gu_pair, 0, unroll=False)

    pltpu.sync_copy(interbuf.at[pl.ds(0, 88)], shared.at[pl.ds(s * 88, 88)])
    plsc.subcore_barrier()
    pltpu.sync_copy(shared, interfull)

    dbufs = (dbuf0, dbuf1)

    def start_dn(k, p):
        kk = jnp.minimum(k, DN_CH - 1)

        @pl.when(k < DN_CH)
        def _():
            pltpu.async_copy(dr.at[div.at[pl.ds(kk * RC, RC)]], dbufs[p], gsems[p])

    start_dn(0, 0)
    start_dn(1, 1)

    def dn_pair(i, _):
        dsc = []
        for p in range(2):
            k = 2 * i + p
            pltpu.make_async_copy(dr.at[div.at[pl.ds(0, RC)]], dbufs[p], gsems[p]).wait()
            dsc += _dot8(dbufs[p], interfull, INTER)
            start_dn(k + 2, p)
        dvec = _insert8(jnp.zeros((L,), jnp.float32), dsc[:8], 0, lane_iota)
        dvec = _insert8(dvec, dsc[8:], 8, lane_iota)
        obuf[pl.ds(i * L, L)] = dvec
        return 0

    lax.fori_loop(0, DN_CH // 2, dn_pair, 0, unroll=False)

    pltpu.sync_copy(obuf, outr.at[c, pl.ds(s * 128, 128)])


def _run_sc(x_flat, gate_rows, up_rows, down_rows, gu_idx, dn_idx, w16):
    mesh = plsc.VectorSubcoreMesh(core_axis_name="c", subcore_axis_name="s")
    fn = pl.kernel(
        _sc_body,
        out_type=jax.ShapeDtypeStruct((2, HIDDEN), jnp.float32),
        mesh=mesh,
        compiler_params=pltpu.CompilerParams(needs_layout_passes=False),
        scratch_types=[
            pltpu.VMEM((HIDDEN,), jnp.float32),
            pltpu.VMEM((88,), jnp.int32),
            pltpu.VMEM((128,), jnp.int32),
            pltpu.VMEM((L,), jnp.float32),
            pltpu.VMEM((RC, HIDDEN), jnp.float32),
            pltpu.VMEM((RC, HIDDEN), jnp.float32),
            pltpu.VMEM((RC, HIDDEN), jnp.float32),
            pltpu.VMEM((RC, HIDDEN), jnp.float32),
            pltpu.VMEM((RC, INTER), jnp.float32),
            pltpu.VMEM((RC, INTER), jnp.float32),
            pltpu.VMEM((96,), jnp.float32),
            pltpu.VMEM((INTER,), jnp.float32),
            pltpu.VMEM((128,), jnp.float32),
            pltpu.VMEM_SHARED((INTER,), jnp.float32),
            pltpu.SemaphoreType.DMA,
            pltpu.SemaphoreType.DMA,
            pltpu.SemaphoreType.DMA,
            pltpu.SemaphoreType.DMA,
        ],
    )
    return fn(x_flat, gate_rows, up_rows, down_rows, gu_idx, dn_idx, w16)


@jax.jit
def _run(x_bc1t, topk_idx, topk_weights, gate_proj_all, up_proj_all, down_proj_all):
    x_flat = x_bc1t.reshape(HIDDEN)
    idx = topk_idx.astype(jnp.int32)
    gu_idx = (idx[:, None] * INTER
              + jnp.arange(INTER, dtype=jnp.int32)[None, :]).reshape(K * INTER)
    dn_idx = (idx[:, None] * HIDDEN
              + jnp.arange(HIDDEN, dtype=jnp.int32)[None, :]).reshape(K * HIDDEN)
    w16 = jnp.zeros((L,), jnp.float32).at[:K].set(topk_weights)
    gate_rows = gate_proj_all.reshape(-1, HIDDEN)
    up_rows = up_proj_all.reshape(-1, HIDDEN)
    down_rows = down_proj_all.reshape(-1, INTER)
    tc_part = _run_tc(x_flat.reshape(1, HIDDEN), idx, topk_weights,
                      gate_proj_all, up_proj_all, down_proj_all)
    sc_part = _run_sc(x_flat, gate_rows, up_rows, down_rows, gu_idx, dn_idx, w16)
    out = tc_part[0] + sc_part[0] + sc_part[1]
    return out.reshape(1, HIDDEN, 1, 1)


def kernel(x_bc1t, topk_idx, topk_weights, gate_proj_all, up_proj_all, down_proj_all):
    return _run(x_bc1t, topk_idx, topk_weights, gate_proj_all,
                up_proj_all, down_proj_all)
